# baseline (speedup 1.0000x reference)
import functools

import jax
import jax.numpy as jnp
from jax import lax
from jax.experimental import pallas as pl
from jax.experimental.pallas import tpu as pltpu
from jax.experimental.pallas import tpu_sc as plsc

N = 10000
E = 320000
D = 128
NC = 2
NS = 16
CHUNK = 128
NCHUNKS = E // CHUNK
ROWS_PER_TILE = 632
WBLKS = (128, 128, 128, 128, 120)
LBLKS = (128, 128, 128, 128, 8)

NRB = 3
NIB = 6
P1_OUTER = NCHUNKS // NS // NIB
P1_XTRA = NCHUNKS - P1_OUTER * NIB * NS
P2_OUTER = NCHUNKS // (NC * NS) // NIB
P2_XTRA = NCHUNKS - P2_OUTER * NIB * NC * NS

_mesh = plsc.VectorSubcoreMesh(
    core_axis_name="c", subcore_axis_name="s", num_cores=NC, num_subcores=NS)

_SCRATCH = (
    [pltpu.VMEM((2, CHUNK), jnp.int32)] * 6
    + [pltpu.VMEM((CHUNK, D), jnp.float32)] * 3
    + [pltpu.VMEM_SHARED((N, D), jnp.float32)]
    + [pltpu.SemaphoreType.DMA] * 12
)


def _make_helpers(scratch):
    (sd0, sd1, sd2, sd3, sd4, sd5, rows0, rows1, rows2, acc,
     isem0, isem1, isem2, isem3, isem4, isem5,
     gsem0, gsem1, gsem2, ssem0, ssem1, ssem2) = scratch
    c = lax.axis_index("c")
    s = lax.axis_index("s")
    wid = c * NS + s
    row0 = s * ROWS_PER_TILE
    sds = (sd0, sd1, sd2, sd3, sd4, sd5)
    rowss = (rows0, rows1, rows2)
    isems = (isem0, isem1, isem2, isem3, isem4, isem5)
    gsems = (gsem0, gsem1, gsem2)
    ssems = (ssem0, ssem1, ssem2)

    def tile_blocks(fn):
        def blast(blks):
            waits = [fn(o, w, isems[i])
                     for i, (o, w) in enumerate(zip(
                         (0, 128, 256, 384, 512), blks))]
            for wt in waits:
                wt()

        @pl.when(s < NS - 1)
        def _():
            blast(WBLKS)

        @pl.when(s == NS - 1)
        def _():
            blast(LBLKS)

    def zero_acc():
        def zrow(i, carry):
            for k in range(D // 16):
                rows0[i, pl.ds(k * 16, 16)] = jnp.zeros((16,), jnp.float32)
            return carry
        lax.fori_loop(0, CHUNK, zrow, 0)

        def zblk(o, w, sem):
            cp = pltpu.async_copy(
                rows0.at[pl.ds(0, w)], acc.at[pl.ds(row0 + o, w)], sem)
            return cp.wait
        tile_blocks(zblk)

    def run_edges(idx_ref, table_ref, first, stride, nouter, nxtra):
        def off(j):
            return (first + j * stride) * CHUNK

        def fire_idx(j, ib):
            pltpu.async_copy(idx_ref.at[:, pl.ds(off(j), CHUNK)], sds[ib], isems[ib])

        def wait_idx(j, ib):
            pltpu.make_async_copy(
                idx_ref.at[:, pl.ds(off(j), CHUNK)], sds[ib], isems[ib]).wait()

        def fire_gather(ib, rb):
            pltpu.async_copy(table_ref.at[sds[ib].at[0]], rowss[rb], gsems[rb])

        def wait_gather(ib, rb):
            pltpu.make_async_copy(table_ref.at[sds[ib].at[0]], rowss[rb], gsems[rb]).wait()

        def fire_scatter(ib, rb):
            pltpu.async_copy(rowss[rb], acc.at[sds[ib].at[1]], ssems[rb], add=True)

        def wait_scatter(ib, rb):
            pltpu.make_async_copy(rowss[rb], acc.at[sds[ib].at[1]], ssems[rb]).wait()

        for j0 in range(4):
            fire_idx(j0, j0)
        wait_idx(0, 0)
        fire_gather(0, 0)

        def outer(t, carry):
            for b in range(NIB):
                j = NIB * t + b
                rb = b % NRB
                if b >= 2:
                    wait_scatter((b - 2) % NIB, (b + 1) % NRB)
                else:
                    @pl.when(t > 0)
                    def _():
                        wait_scatter((b - 2) % NIB, (b + 1) % NRB)
                if b < NIB - 1:
                    wait_idx(j + 1, (b + 1) % NIB)
                    fire_gather((b + 1) % NIB, (b + 1) % NRB)
                else:
                    @pl.when(t < nouter - 1)
                    def _():
                        wait_idx(j + 1, 0)
                        fire_gather(0, 0)
                if b < 2:
                    fire_idx(j + 4, (b + 4) % NIB)
                else:
                    @pl.when(t < nouter - 1)
                    def _():
                        fire_idx(j + 4, (b + 4) % NIB)
                wait_gather(b, rb)
                fire_scatter(b, rb)
            return carry

        lax.fori_loop(0, nouter, outer, 0)
        wait_scatter(NIB - 2, (NIB - 2) % NRB)
        wait_scatter(NIB - 1, (NIB - 1) % NRB)

        me = s if stride == NS else wid

        @pl.when(me < nxtra)
        def _():
            o = (nouter * NIB * stride + me) * CHUNK
            pltpu.sync_copy(idx_ref.at[:, pl.ds(o, CHUNK)], sd0)
            pltpu.async_copy(table_ref.at[sd0.at[0]], rows0, gsem0).wait()
            pltpu.async_copy(rows0, acc.at[sd0.at[1]], ssem0, add=True)
            pltpu.make_async_copy(rows0, acc.at[sd0.at[1]], ssem0).wait()

    def write_out(dst_hbm, dst_base):
        def wblk(o, w, sem):
            cp = pltpu.async_copy(
                acc.at[pl.ds(row0 + o, w)],
                dst_hbm.at[pl.ds(dst_base + row0 + o, w)], sem)
            return cp.wait
        tile_blocks(wblk)

    return c, s, wid, zero_acc, run_edges, write_out


def _ud_body(x, up, down, out_up, out_down, *scratch):
    c, s, wid, zero_acc, run_edges, write_out = _make_helpers(scratch)
    zero_acc()
    plsc.subcore_barrier()

    @pl.when(c == 0)
    def _():
        run_edges(up, x, s, NS, P1_OUTER, P1_XTRA)

    @pl.when(c == 1)
    def _():
        run_edges(down, x, s, NS, P1_OUTER, P1_XTRA)

    plsc.subcore_barrier()

    @pl.when(c == 0)
    def _():
        write_out(out_up, 0)

    @pl.when(c == 1)
    def _():
        write_out(out_down, 0)


def _b_body(battr, bnd, pb, *scratch):
    c, s, wid, zero_acc, run_edges, write_out = _make_helpers(scratch)
    zero_acc()
    plsc.subcore_barrier()
    run_edges(bnd, battr, wid, NC * NS, P2_OUTER, P2_XTRA)
    plsc.subcore_barrier()
    write_out(pb, c * N)


_ud_call = pl.kernel(
    _ud_body,
    out_type=[
        jax.ShapeDtypeStruct((N, D), jnp.float32),
        jax.ShapeDtypeStruct((N, D), jnp.float32),
    ],
    mesh=_mesh,
    scratch_types=list(_SCRATCH),
)

_b_call = pl.kernel(
    _b_body,
    out_type=[jax.ShapeDtypeStruct((2 * N, D), jnp.float32)],
    mesh=_mesh,
    scratch_types=list(_SCRATCH),
)

_BLK = 1000


def _add_body(a_ref, b_ref, o_ref):
    o_ref[...] = a_ref[...] + b_ref[...]


_tc_add = pl.pallas_call(
    _add_body,
    grid=(N // _BLK,),
    in_specs=[
        pl.BlockSpec((_BLK, D), lambda g: (g, 0)),
        pl.BlockSpec((_BLK, D), lambda g: (g + N // _BLK, 0)),
    ],
    out_specs=pl.BlockSpec((_BLK, D), lambda g: (g, 0)),
    out_shape=jax.ShapeDtypeStruct((N, D), jnp.float32),
)


@jax.jit
def kernel(x, up_index, down_index, boundary_index, boundary_attr):
    (pbp,) = _b_call(boundary_attr, boundary_index)
    out_boundary = _tc_add(pbp, pbp)
    out_up, out_down = _ud_call(x, up_index, down_index)
    return (out_up, out_down, out_boundary)


# speedup vs baseline: 1.0001x; 1.0001x over previous
import jax
import jax.numpy as jnp
from jax import lax
from jax.experimental import pallas as pl
from jax.experimental.pallas import tpu as pltpu
from jax.experimental.pallas import tpu_sc as plsc

N = 10000
E = 320000
D = 128
NC = 2
NS = 16
CHUNK = 128
NCHUNKS = E // CHUNK
ROWS_PER_TILE = 632
WBLKS = (128, 128, 128, 128, 120)
LBLKS = (128, 128, 128, 128, 8)

NRB = 3
NIB = 6
P1_OUTER = NCHUNKS // NS // NIB
P1_XTRA = NCHUNKS - P1_OUTER * NIB * NS
P2_OUTER = NCHUNKS // (NC * NS) // NIB
P2_XTRA = NCHUNKS - P2_OUTER * NIB * NC * NS

_mesh = plsc.VectorSubcoreMesh(
    core_axis_name="c", subcore_axis_name="s", num_cores=NC, num_subcores=NS)

_SCRATCH = (
    [pltpu.VMEM((2, CHUNK), jnp.int32)] * 6
    + [pltpu.VMEM((CHUNK, D), jnp.float32)] * 3
    + [pltpu.VMEM_SHARED((N, D), jnp.float32)]
    + [pltpu.SemaphoreType.DMA] * 12
)


def _make_helpers(scratch):
    (sd0, sd1, sd2, sd3, sd4, sd5, rows0, rows1, rows2, acc,
     isem0, isem1, isem2, isem3, isem4, isem5,
     gsem0, gsem1, gsem2, ssem0, ssem1, ssem2) = scratch
    c = lax.axis_index("c")
    s = lax.axis_index("s")
    wid = c * NS + s
    row0 = s * ROWS_PER_TILE
    sds = (sd0, sd1, sd2, sd3, sd4, sd5)
    rowss = (rows0, rows1, rows2)
    isems = (isem0, isem1, isem2, isem3, isem4, isem5)
    gsems = (gsem0, gsem1, gsem2)
    ssems = (ssem0, ssem1, ssem2)

    def tile_blocks(fn):
        def blast(blks):
            waits = [fn(o, w, isems[i])
                     for i, (o, w) in enumerate(zip(
                         (0, 128, 256, 384, 512), blks))]
            for wt in waits:
                wt()

        @pl.when(s < NS - 1)
        def _():
            blast(WBLKS)

        @pl.when(s == NS - 1)
        def _():
            blast(LBLKS)

    def zero_acc():
        def zrow(i, carry):
            for k in range(D // 16):
                rows0[i, pl.ds(k * 16, 16)] = jnp.zeros((16,), jnp.float32)
            return carry
        lax.fori_loop(0, CHUNK, zrow, 0)

        def zblk(o, w, sem):
            cp = pltpu.async_copy(
                rows0.at[pl.ds(0, w)], acc.at[pl.ds(row0 + o, w)], sem)
            return cp.wait
        tile_blocks(zblk)

    def run_edges(idx_ref, table_ref, first, stride, nouter, nxtra):
        def off(j):
            return (first + j * stride) * CHUNK

        def fire_idx(j, ib):
            pltpu.async_copy(idx_ref.at[:, pl.ds(off(j), CHUNK)], sds[ib], isems[ib])

        def wait_idx(j, ib):
            pltpu.make_async_copy(
                idx_ref.at[:, pl.ds(off(j), CHUNK)], sds[ib], isems[ib]).wait()

        def fire_gather(ib, rb):
            pltpu.async_copy(table_ref.at[sds[ib].at[0]], rowss[rb], gsems[rb])

        def wait_gather(ib, rb):
            pltpu.make_async_copy(table_ref.at[sds[ib].at[0]], rowss[rb], gsems[rb]).wait()

        def fire_scatter(ib, rb):
            pltpu.async_copy(rowss[rb], acc.at[sds[ib].at[1]], ssems[rb], add=True)

        def wait_scatter(ib, rb):
            pltpu.make_async_copy(rowss[rb], acc.at[sds[ib].at[1]], ssems[rb]).wait()

        for j0 in range(4):
            fire_idx(j0, j0)
        wait_idx(0, 0)
        fire_gather(0, 0)

        def outer(t, carry):
            for b in range(NIB):
                j = NIB * t + b
                rb = b % NRB
                if b >= 2:
                    wait_scatter((b - 2) % NIB, (b + 1) % NRB)
                else:
                    @pl.when(t > 0)
                    def _():
                        wait_scatter((b - 2) % NIB, (b + 1) % NRB)
                if b < NIB - 1:
                    wait_idx(j + 1, (b + 1) % NIB)
                    fire_gather((b + 1) % NIB, (b + 1) % NRB)
                else:
                    @pl.when(t < nouter - 1)
                    def _():
                        wait_idx(j + 1, 0)
                        fire_gather(0, 0)
                if b < 2:
                    fire_idx(j + 4, (b + 4) % NIB)
                else:
                    @pl.when(t < nouter - 1)
                    def _():
                        fire_idx(j + 4, (b + 4) % NIB)
                wait_gather(b, rb)
                fire_scatter(b, rb)
            return carry

        lax.fori_loop(0, nouter, outer, 0)
        wait_scatter(NIB - 2, (NIB - 2) % NRB)
        wait_scatter(NIB - 1, (NIB - 1) % NRB)

        me = s if stride == NS else wid

        @pl.when(me < nxtra)
        def _():
            o = (nouter * NIB * stride + me) * CHUNK
            pltpu.sync_copy(idx_ref.at[:, pl.ds(o, CHUNK)], sd0)
            pltpu.async_copy(table_ref.at[sd0.at[0]], rows0, gsem0).wait()
            pltpu.async_copy(rows0, acc.at[sd0.at[1]], ssem0, add=True)
            pltpu.make_async_copy(rows0, acc.at[sd0.at[1]], ssem0).wait()

    def write_out(dst_hbm, dst_base):
        def wblk(o, w, sem):
            cp = pltpu.async_copy(
                acc.at[pl.ds(row0 + o, w)],
                dst_hbm.at[pl.ds(dst_base + row0 + o, w)], sem)
            return cp.wait
        tile_blocks(wblk)

    return c, s, wid, zero_acc, run_edges, write_out


def _ud_body(x, up, down, out_up, out_down, *scratch):
    c, s, wid, zero_acc, run_edges, write_out = _make_helpers(scratch)
    zero_acc()
    plsc.subcore_barrier()

    @pl.when(c == 0)
    def _():
        run_edges(up, x, s, NS, P1_OUTER, P1_XTRA)

    @pl.when(c == 1)
    def _():
        run_edges(down, x, s, NS, P1_OUTER, P1_XTRA)

    plsc.subcore_barrier()

    @pl.when(c == 0)
    def _():
        write_out(out_up, 0)

    @pl.when(c == 1)
    def _():
        write_out(out_down, 0)


def _b_body(battr, bnd, pb, *scratch):
    c, s, wid, zero_acc, run_edges, write_out = _make_helpers(scratch)
    zero_acc()
    plsc.subcore_barrier()
    run_edges(bnd, battr, wid, NC * NS, P2_OUTER, P2_XTRA)
    plsc.subcore_barrier()
    write_out(pb, c * N)


_ud_call = pl.kernel(
    _ud_body,
    out_type=[
        jax.ShapeDtypeStruct((N, D), jnp.float32),
        jax.ShapeDtypeStruct((N, D), jnp.float32),
    ],
    mesh=_mesh,
    scratch_types=list(_SCRATCH),
)

_b_call = pl.kernel(
    _b_body,
    out_type=[jax.ShapeDtypeStruct((2 * N, D), jnp.float32)],
    mesh=_mesh,
    scratch_types=list(_SCRATCH),
)

_BLK = 1000


def _add_body(a_ref, b_ref, o_ref):
    o_ref[...] = a_ref[...] + b_ref[...]


_tc_add = pl.pallas_call(
    _add_body,
    grid=(N // _BLK,),
    in_specs=[
        pl.BlockSpec((_BLK, D), lambda g: (g, 0)),
        pl.BlockSpec((_BLK, D), lambda g: (g + N // _BLK, 0)),
    ],
    out_specs=pl.BlockSpec((_BLK, D), lambda g: (g, 0)),
    out_shape=jax.ShapeDtypeStruct((N, D), jnp.float32),
)


@jax.jit
def kernel(x, up_index, down_index, boundary_index, boundary_attr):
    (pbp,) = _b_call(boundary_attr, boundary_index)
    out_boundary = _tc_add(pbp, pbp)
    out_up, out_down = _ud_call(x, up_index, down_index)
    return (out_up, out_down, out_boundary)
